# baseline (speedup 1.0000x reference)
import functools

import jax
import jax.numpy as jnp
from jax import lax
from jax.experimental import pallas as pl
from jax.experimental.pallas import tpu as pltpu
from jax.experimental.pallas import tpu_sc as plsc

B, S, D = 4, 8192, 2048
NC, NS = 2, 16
NW = NC * NS
ROWS_PER_W = S // NW
CH = 16
NCH = ROWS_PER_W // CH

_mesh = plsc.VectorSubcoreMesh(core_axis_name="c", subcore_axis_name="s")


@functools.partial(
    pl.kernel,
    mesh=_mesh,
    out_type=jax.ShapeDtypeStruct((B * S, D), jnp.float32),
    scratch_types=[
        pltpu.VMEM((CH, D), jnp.float32),
        pltpu.VMEM((CH, D), jnp.float32),
        pltpu.SemaphoreType.DMA,
        pltpu.SemaphoreType.DMA,
        pltpu.SemaphoreType.DMA,
        pltpu.SemaphoreType.DMA,
    ],
)
def _bcast_sc(pos_hbm, out_hbm, buf0, buf1, g0, g1, s0, s1):
    wid = lax.axis_index("s") * NC + lax.axis_index("c")
    base0 = wid * ROWS_PER_W
    bufs, gsems, ssems = [buf0, buf1], [g0, g1], [s0, s1]

    def gather(c):
        i = c % 2
        return pltpu.async_copy(
            pos_hbm.at[pl.ds(base0 + c * CH, CH)], bufs[i], gsems[i])

    def scatters(c):
        i = c % 2
        return [
            pltpu.async_copy(
                bufs[i], out_hbm.at[pl.ds(b * S + base0 + c * CH, CH)],
                ssems[i])
            for b in range(B)
        ]

    pending_g = gather(0)
    pending_s = {}
    for c in range(NCH):
        pending_g.wait()
        pending_s[c] = scatters(c)
        if c + 1 < NCH:
            if c >= 1:
                for cp in pending_s.pop(c - 1):
                    cp.wait()
            pending_g = gather(c + 1)
    for c in sorted(pending_s):
        for cp in pending_s[c]:
            cp.wait()


def kernel(x, pos_emb):
    del x
    out2d = _bcast_sc(pos_emb)
    return out2d.reshape(B, S, D)


# speedup vs baseline: 1.0506x; 1.0506x over previous
import functools

import jax
import jax.numpy as jnp
from jax import lax
from jax.experimental import pallas as pl
from jax.experimental.pallas import tpu as pltpu
from jax.experimental.pallas import tpu_sc as plsc

B, S, D = 4, 8192, 2048
NC, NS = 2, 16
NW = NC * NS
ROWS_PER_W = S // NW

CHUNKS = [32, 24, 32, 24, 32, 24, 32, 24, 32]
OFFS = [sum(CHUNKS[:i]) for i in range(len(CHUNKS))]
NCH = len(CHUNKS)
B0, B1 = 32, 24

_mesh = plsc.VectorSubcoreMesh(core_axis_name="c", subcore_axis_name="s")


@functools.partial(
    pl.kernel,
    mesh=_mesh,
    out_type=jax.ShapeDtypeStruct((B * S, D), jnp.float32),
    scratch_types=[
        pltpu.VMEM((B0, D), jnp.float32),
        pltpu.VMEM((B1, D), jnp.float32),
        pltpu.SemaphoreType.DMA,
        pltpu.SemaphoreType.DMA,
        pltpu.SemaphoreType.DMA,
        pltpu.SemaphoreType.DMA,
    ],
)
def _bcast_sc(pos_hbm, out_hbm, buf0, buf1, g0, g1, s0, s1):
    wid = lax.axis_index("s") * NC + lax.axis_index("c")
    base0 = wid * ROWS_PER_W
    bufs, gsems, ssems = [buf0, buf1], [g0, g1], [s0, s1]

    def buf_for(c):
        i = c % 2
        n = CHUNKS[c]
        b = bufs[i]
        return b if n == b.shape[0] else b.at[pl.ds(0, n)]

    def gather(c):
        return pltpu.async_copy(
            pos_hbm.at[pl.ds(base0 + OFFS[c], CHUNKS[c])],
            buf_for(c), gsems[c % 2])

    def scatters(c):
        return [
            pltpu.async_copy(
                buf_for(c),
                out_hbm.at[pl.ds(b * S + base0 + OFFS[c], CHUNKS[c])],
                ssems[c % 2])
            for b in range(B)
        ]

    pending_g = gather(0)
    pending_s = {}
    for c in range(NCH):
        pending_g.wait()
        pending_s[c] = scatters(c)
        if c + 1 < NCH:
            if c >= 1:
                for cp in pending_s.pop(c - 1):
                    cp.wait()
            pending_g = gather(c + 1)
    for c in sorted(pending_s):
        for cp in pending_s[c]:
            cp.wait()


def kernel(x, pos_emb):
    del x
    out2d = _bcast_sc(pos_emb)
    return out2d.reshape(B, S, D)


# speedup vs baseline: 1.3534x; 1.2882x over previous
import functools

import jax
import jax.numpy as jnp
from jax.experimental import pallas as pl
from jax.experimental.pallas import tpu as pltpu

B, S, D = 4, 8192, 2048
BS = 256


def _body(in_ref, out_ref):
    out_ref[...] = jnp.broadcast_to(in_ref[...][None], (B, BS, D))


def kernel(x, pos_emb):
    del x
    return pl.pallas_call(
        _body,
        grid=(S // BS,),
        in_specs=[pl.BlockSpec((BS, D), lambda i: (i, 0))],
        out_specs=pl.BlockSpec((B, BS, D), lambda i: (0, i, 0)),
        out_shape=jax.ShapeDtypeStruct((B, S, D), jnp.float32),
    )(pos_emb)
